# baseline (device time: 9601 ns/iter reference)
import jax
import jax.numpy as jnp
from jax import lax
from jax.experimental import pallas as pl
from jax.experimental.pallas import tpu as pltpu

NCHUNK = 2


def kernel(x, W, labels):
    T, D = x.shape
    _, Vs = W.shape
    CK = Vs // NCHUNK

    def body(x_hbm, w_hbm, lab_hbm, out_ref,
             x_vmem, w_vmem, lab_vmem, send_buf, recv_buf, nll_vmem,
             in_sems, chunk_sems, send_sem, recv_sem, out_sem):
        my_x = lax.axis_index("x")
        my_y = lax.axis_index("y")
        peer = (my_x, 1 - my_y)

        x_cp = pltpu.make_async_copy(x_hbm, x_vmem, in_sems.at[0])
        lab_cp = pltpu.make_async_copy(lab_hbm, lab_vmem, in_sems.at[1])
        x_cp.start()
        lab_cp.start()
        w_cps = []
        for c in range(NCHUNK):
            cp = pltpu.make_async_copy(
                w_hbm.at[:, pl.ds(c * CK, CK)],
                w_vmem.at[:, pl.ds(c * CK, CK)],
                chunk_sems.at[c],
            )
            cp.start()
            w_cps.append(cp)

        barrier_sem = pltpu.get_barrier_semaphore()
        pl.semaphore_signal(
            barrier_sem, inc=1,
            device_id=peer, device_id_type=pl.DeviceIdType.MESH,
        )

        x_cp.wait()
        lab_cp.wait()
        xv = x_vmem[:, :]
        lab_row = lab_vmem[:, :]

        s = None
        ll = None
        for c in range(NCHUNK):
            w_cps[c].wait()
            chunkT = lax.dot_general(
                w_vmem[:, c * CK:(c + 1) * CK], xv,
                ((( 0,), (1,)), ((), ())),
                preferred_element_type=jnp.float32)
            ids = (lax.broadcasted_iota(jnp.int32, (CK, T), 0)
                   + (my_y * Vs + c * CK))
            cs = jnp.sum(jnp.exp(chunkT), axis=0, keepdims=True)
            cll = jnp.sum(jnp.where(ids == lab_row, chunkT, 0.0),
                          axis=0, keepdims=True)
            s = cs if s is None else s + cs
            ll = cll if ll is None else ll + cll

        send_buf[0:1, :] = s
        send_buf[1:2, :] = ll

        pl.semaphore_wait(barrier_sem, 1)

        rdma = pltpu.make_async_remote_copy(
            src_ref=send_buf,
            dst_ref=recv_buf,
            send_sem=send_sem,
            recv_sem=recv_sem,
            device_id=peer,
            device_id_type=pl.DeviceIdType.MESH,
        )
        rdma.start()
        rdma.wait()

        s_o = recv_buf[0:1, :]
        ll_o = recv_buf[1:2, :]
        nll_vmem[:, :] = jnp.log(s + s_o) - (ll + ll_o)
        out_cp = pltpu.make_async_copy(nll_vmem, out_ref, out_sem)
        out_cp.start()
        out_cp.wait()

    hbm = pltpu.MemorySpace.HBM
    out = pl.pallas_call(
        body,
        out_shape=jax.ShapeDtypeStruct((1, T), jnp.float32),
        in_specs=[
            pl.BlockSpec(memory_space=hbm),
            pl.BlockSpec(memory_space=hbm),
            pl.BlockSpec(memory_space=hbm),
        ],
        out_specs=pl.BlockSpec(memory_space=hbm),
        scratch_shapes=[
            pltpu.VMEM((T, D), jnp.float32),
            pltpu.VMEM((D, Vs), jnp.float32),
            pltpu.VMEM((1, T), jnp.int32),
            pltpu.VMEM((2, T), jnp.float32),
            pltpu.VMEM((2, T), jnp.float32),
            pltpu.VMEM((1, T), jnp.float32),
            pltpu.SemaphoreType.DMA((2,)),
            pltpu.SemaphoreType.DMA((NCHUNK,)),
            pltpu.SemaphoreType.DMA,
            pltpu.SemaphoreType.DMA,
            pltpu.SemaphoreType.DMA,
        ],
        compiler_params=pltpu.CompilerParams(collective_id=0),
    )(
        pltpu.with_memory_space_constraint(x, hbm),
        pltpu.with_memory_space_constraint(W, hbm),
        pltpu.with_memory_space_constraint(labels.reshape(1, T), hbm),
    )
    return out.reshape(T)
